# baseline (speedup 1.0000x reference)
import functools

import jax
import jax.numpy as jnp
from jax import lax
from jax.experimental import pallas as pl
from jax.experimental.pallas import tpu as pltpu
from jax.experimental.pallas import tpu_sc as plsc

N = 50000
C = 256
B = 1024
STEPS = 4

NC = 2
NS = 16
NW = NC * NS
L = 16
CL = C // L

RPT = 1664
NP = NW * RPT
CH = 128
NCHUNK = RPT // CH
NB = 2 * NW
NEG = -1.0e30


def _sigmoid(z):
    return 1.0 / (1.0 + jnp.exp(-z))


def _store1(ref, idx, val):
    lane0 = lax.broadcasted_iota(jnp.int32, (L,), 0) == 0
    plsc.store_scatter(
        ref,
        [jnp.broadcast_to(idx, (L,)).astype(jnp.int32)],
        jnp.broadcast_to(val, (L,)),
        mask=lane0,
    )


def _build_r(rnum, dpart_flat, bmd, bacc):
    rn = rnum[0] + rnum[1]
    d = jnp.sum(dpart_flat, axis=0)
    meta = bmd
    bm = meta[:, 0]
    bd = meta[:, 1]
    bseg = meta[:, 2].astype(jnp.int32)
    ids = lax.broadcasted_iota(jnp.int32, (B, NB), 0)
    onehot = (ids == bseg[None, :]) & (bseg[None, :] >= 0)
    mmax = jnp.max(jnp.where(onehot, bm[None, :], NEG), axis=1)
    mmax_j = jnp.sum(jnp.where(onehot, mmax[:, None], 0.0), axis=0)
    w = jnp.where(bseg >= 0, jnp.exp(bm - mmax_j), 0.0)
    oh_f = onehot.astype(jnp.float32)
    db = jnp.sum(oh_f * (bd * w)[None, :], axis=1)
    rb = jnp.dot(oh_f, bacc * w[:, None],
                 preferred_element_type=jnp.float32)
    return (rn + rb) / (d + db + 1e-16)[:, None]


def _lstm_body(q_ref, rnum_ref, d_ref, bmd_ref, bacc_ref, c_ref,
               wih_ref, whh_ref, b_ref, qo_ref, co_ref):
    q_prev = q_ref[...]
    r = _build_r(rnum_ref[...], d_ref[...], bmd_ref[...], bacc_ref[...])
    wih = wih_ref[...]
    gates = (
        jnp.dot(q_prev, wih[:C, :], preferred_element_type=jnp.float32)
        + jnp.dot(r, wih[C:, :], preferred_element_type=jnp.float32)
        + jnp.dot(q_prev, whh_ref[...], preferred_element_type=jnp.float32)
        + b_ref[...]
    )
    gi = gates[:, 0:C]
    gf = gates[:, C:2 * C]
    gg = gates[:, 2 * C:3 * C]
    go = gates[:, 3 * C:4 * C]
    c_new = _sigmoid(gf) * c_ref[...] + _sigmoid(gi) * jnp.tanh(gg)
    qo_ref[...] = _sigmoid(go) * jnp.tanh(c_new)
    co_ref[...] = c_new


def _lstm_step(q, rnum, dpart, bmd, bacc, c, wih_t, whh_t, bias):
    return pl.pallas_call(
        _lstm_body,
        out_shape=[
            jax.ShapeDtypeStruct((B, C), jnp.float32),
            jax.ShapeDtypeStruct((B, C), jnp.float32),
        ],
    )(q, rnum, dpart, bmd, bacc, c, wih_t, whh_t, bias)


def _final_body(q_ref, rnum_ref, d_ref, bmd_ref, bacc_ref, out_ref):
    out_ref[:, :C] = q_ref[...]
    out_ref[:, C:] = _build_r(rnum_ref[...], d_ref[...], bmd_ref[...],
                              bacc_ref[...])


def _final(q, rnum, dpart, bmd, bacc):
    return pl.pallas_call(
        _final_body,
        out_shape=jax.ShapeDtypeStruct((B, 2 * C), jnp.float32),
    )(q, rnum, dpart, bmd, bacc)


_MESH = plsc.VectorSubcoreMesh(
    core_axis_name="c", subcore_axis_name="s", num_cores=NC, num_subcores=NS)


def _att_body(x_hbm, q_hbm, seg_hbm, dpart_hbm, rnum_hbm, bmd_hbm, bacc_hbm,
              segc_v, qg_v, xc_v, d_v, accbuf, b0md, b0acc, md_st, idx1,
              zbuf, shared, sem):
    cid = lax.axis_index("c")
    sid = lax.axis_index("s")
    wid = sid * NC + cid
    base = wid * RPT
    zero16 = jnp.zeros((L,), jnp.float32)

    def zrow(i, _):
        def zcol(j, _):
            zbuf[i, pl.ds(j * L, L)] = zero16
            return 0
        lax.fori_loop(0, CL, zcol, 0)
        return 0

    lax.fori_loop(0, CH, zrow, 0)

    @pl.when(sid == 0)
    def _():
        def zshared(j, _):
            pltpu.sync_copy(zbuf, shared.at[pl.ds(j * CH, CH)])
            return 0
        lax.fori_loop(0, B // CH, zshared, 0)

    plsc.subcore_barrier()

    def dinit(i, _):
        d_v[pl.ds(i * L, L)] = zero16
        return 0

    lax.fori_loop(0, (B + L) // L, dinit, 0)

    def zsmall(i, _):
        b0acc[0, pl.ds(i * L, L)] = zero16
        accbuf[0, pl.ds(i * L, L)] = zero16
        return 0

    lax.fori_loop(0, CL, zsmall, 0)
    b0md[pl.ds(0, L)] = zero16
    _store1(b0md, 0, jnp.float32(NEG))
    _store1(b0md, 2, jnp.float32(-1.0))

    def chunk_body(k, carry):
        row0 = base + k * CH
        pltpu.sync_copy(seg_hbm.at[pl.ds(row0, CH)], segc_v)
        pltpu.sync_copy(x_hbm.at[pl.ds(row0, CH)], xc_v)
        pltpu.async_copy(q_hbm.at[segc_v], qg_v, sem).wait()

        ngrp = jnp.clip(N - row0, 0, CH) // L

        def grp_body(gi, carry):
            cur_seg, m, dd, first_done, *acc = carry
            off = gi * L
            sv = segc_v[pl.ds(off, L)]
            for j in range(L):
                s = sv[j]
                rr = off + j

                xr = [xc_v[rr, pl.ds(t * L, L)] for t in range(CL)]
                p = [zero16, zero16, zero16, zero16]
                for t in range(CL):
                    p[t % 4] = p[t % 4] + xr[t] * qg_v[rr, pl.ds(t * L, L)]
                e = jnp.sum((p[0] + p[1]) + (p[2] + p[3]))

                new = s != cur_seg
                flush = jnp.logical_and(new, cur_seg >= 0)

                @pl.when(flush)
                def _():
                    @pl.when(first_done == 0)
                    def _():
                        for t in range(CL):
                            b0acc[0, pl.ds(t * L, L)] = acc[t]
                        _store1(b0md, 0, m)
                        _store1(b0md, 1, dd)
                        _store1(b0md, 2, cur_seg.astype(jnp.float32))

                    @pl.when(first_done != 0)
                    def _():
                        for t in range(CL):
                            accbuf[0, pl.ds(t * L, L)] = acc[t]
                        _store1(d_v, cur_seg, dd)
                        _store1(idx1, 0, cur_seg)
                        pltpu.sync_copy(accbuf, shared.at[idx1], add=True)

                first_done = jnp.where(flush, 1, first_done)

                m_new = jnp.where(new, e, jnp.maximum(m, e))
                scale = jnp.exp(jnp.broadcast_to(
                    jnp.where(new, NEG, m - m_new), (L,)))
                ee = jnp.exp(jnp.broadcast_to(e - m_new, (L,)))
                dd = jnp.where(new, 1.0, dd * scale[0] + ee[0])
                acc = [acc[t] * scale + ee * xr[t] for t in range(CL)]
                m = m_new
                cur_seg = s
            return (cur_seg, m, dd, first_done, *acc)

        return lax.fori_loop(0, ngrp, grp_body, carry)

    init = (jnp.int32(-1), jnp.float32(NEG), jnp.float32(0.0), jnp.int32(0),
            *([jnp.zeros((L,), jnp.float32)] * CL))
    cur_seg, m, dd, first_done, *acc = lax.fori_loop(
        0, NCHUNK, chunk_body, init)

    pltpu.sync_copy(b0md, bmd_hbm.at[2 * wid])
    pltpu.sync_copy(b0acc, bacc_hbm.at[pl.ds(2 * wid, 1)])

    for t in range(CL):
        accbuf[0, pl.ds(t * L, L)] = acc[t]
    md_st[pl.ds(0, L)] = zero16
    _store1(md_st, 0, m)
    _store1(md_st, 1, dd)
    _store1(md_st, 2, cur_seg.astype(jnp.float32))
    pltpu.sync_copy(md_st, bmd_hbm.at[2 * wid + 1])
    pltpu.sync_copy(accbuf, bacc_hbm.at[pl.ds(2 * wid + 1, 1)])

    pltpu.sync_copy(d_v.at[pl.ds(0, B)], dpart_hbm.at[wid])
    plsc.subcore_barrier()

    @pl.when(sid == 0)
    def _():
        pltpu.sync_copy(shared, rnum_hbm.at[cid])


@functools.partial(
    pl.kernel,
    out_type=[
        jax.ShapeDtypeStruct((NW, B), jnp.float32),
        jax.ShapeDtypeStruct((NC, B, C), jnp.float32),
        jax.ShapeDtypeStruct((NB, L), jnp.float32),
        jax.ShapeDtypeStruct((NB, C), jnp.float32),
    ],
    mesh=_MESH,
    compiler_params=pltpu.CompilerParams(
        use_tc_tiling_on_sc=False, needs_layout_passes=False),
    scratch_types=[
        pltpu.VMEM((CH,), jnp.int32),
        pltpu.VMEM((CH, C), jnp.float32),
        pltpu.VMEM((CH, C), jnp.float32),
        pltpu.VMEM((B + L,), jnp.float32),
        pltpu.VMEM((1, C), jnp.float32),
        pltpu.VMEM((L,), jnp.float32),
        pltpu.VMEM((1, C), jnp.float32),
        pltpu.VMEM((L,), jnp.float32),
        pltpu.VMEM((1,), jnp.int32),
        pltpu.VMEM((CH, C), jnp.float32),
        pltpu.VMEM_SHARED((B, C), jnp.float32),
        pltpu.SemaphoreType.DMA,
    ],
)
def _att_kernel(x_hbm, q_hbm, seg_hbm, dpart_hbm, rnum_hbm, bmd_hbm,
                bacc_hbm, *scratch):
    _att_body(x_hbm, q_hbm, seg_hbm, dpart_hbm, rnum_hbm, bmd_hbm, bacc_hbm,
              *scratch)


def kernel(x, batch, W_ih, W_hh, b_ih, b_hh):
    seg = batch.astype(jnp.int32)
    segp = jnp.pad(seg, (0, NP - N))
    xp = jnp.pad(x, ((0, NP - N), (0, 0)))
    wih_t = W_ih.T
    whh_t = W_hh.T
    bias = (b_ih + b_hh)[None, :]

    q = jnp.zeros((B, C), jnp.float32)
    c = jnp.zeros((B, C), jnp.float32)
    rnum = jnp.zeros((NC, B, C), jnp.float32)
    dpart = jnp.zeros((NW, B), jnp.float32)
    bmd = jnp.zeros((NB, L), jnp.float32).at[:, 2].set(-1.0)
    bacc = jnp.zeros((NB, C), jnp.float32)

    for _ in range(STEPS):
        q, c = _lstm_step(q, rnum, dpart, bmd, bacc, c, wih_t, whh_t, bias)
        dpart, rnum, bmd, bacc = _att_kernel(xp, q, segp)

    return _final(q, rnum, dpart, bmd, bacc)


# speedup vs baseline: 1.2415x; 1.2415x over previous
import functools

import jax
import jax.numpy as jnp
from jax import lax
from jax.experimental import pallas as pl
from jax.experimental.pallas import tpu as pltpu
from jax.experimental.pallas import tpu_sc as plsc

N = 50000
C = 256
B = 1024
STEPS = 4

NC = 2
NS = 16
NW = NC * NS
L = 16
CL = C // L

RPT = 1600
NP = NW * RPT
CH = 80
NCHUNK = RPT // CH
NB = 2 * NW
NEG = -1.0e30


def _sigmoid(z):
    return 1.0 / (1.0 + jnp.exp(-z))


def _store1(ref, idx, val):
    lane0 = lax.broadcasted_iota(jnp.int32, (L,), 0) == 0
    plsc.store_scatter(
        ref,
        [jnp.broadcast_to(idx, (L,)).astype(jnp.int32)],
        jnp.broadcast_to(val, (L,)),
        mask=lane0,
    )


def _build_r(rnum, dpart_flat, bmd, bacc):
    rn = rnum[0] + rnum[1]
    d = jnp.sum(dpart_flat, axis=0)
    meta = bmd
    bm = meta[:, 0]
    bd = meta[:, 1]
    bseg = meta[:, 2].astype(jnp.int32)
    ids = lax.broadcasted_iota(jnp.int32, (B, NB), 0)
    onehot = (ids == bseg[None, :]) & (bseg[None, :] >= 0)
    mmax = jnp.max(jnp.where(onehot, bm[None, :], NEG), axis=1)
    mmax_j = jnp.sum(jnp.where(onehot, mmax[:, None], 0.0), axis=0)
    w = jnp.where(bseg >= 0, jnp.exp(bm - mmax_j), 0.0)
    oh_f = onehot.astype(jnp.float32)
    db = jnp.sum(oh_f * (bd * w)[None, :], axis=1)
    rb = jnp.dot(oh_f, bacc * w[:, None],
                 preferred_element_type=jnp.float32)
    return (rn + rb) / (d + db + 1e-16)[:, None]


def _lstm_body(q_ref, rnum_ref, d_ref, bmd_ref, bacc_ref, c_ref,
               wih_ref, whh_ref, b_ref, qo_ref, co_ref):
    q_prev = q_ref[...]
    r = _build_r(rnum_ref[...], d_ref[...], bmd_ref[...], bacc_ref[...])
    wih = wih_ref[...]
    gates = (
        jnp.dot(q_prev, wih[:C, :], preferred_element_type=jnp.float32)
        + jnp.dot(r, wih[C:, :], preferred_element_type=jnp.float32)
        + jnp.dot(q_prev, whh_ref[...], preferred_element_type=jnp.float32)
        + b_ref[...]
    )
    gi = gates[:, 0:C]
    gf = gates[:, C:2 * C]
    gg = gates[:, 2 * C:3 * C]
    go = gates[:, 3 * C:4 * C]
    c_new = _sigmoid(gf) * c_ref[...] + _sigmoid(gi) * jnp.tanh(gg)
    qo_ref[...] = _sigmoid(go) * jnp.tanh(c_new)
    co_ref[...] = c_new


def _lstm_step(q, rnum, dpart, bmd, bacc, c, wih_t, whh_t, bias):
    return pl.pallas_call(
        _lstm_body,
        out_shape=[
            jax.ShapeDtypeStruct((B, C), jnp.float32),
            jax.ShapeDtypeStruct((B, C), jnp.float32),
        ],
    )(q, rnum, dpart, bmd, bacc, c, wih_t, whh_t, bias)


def _final_body(q_ref, rnum_ref, d_ref, bmd_ref, bacc_ref, out_ref):
    out_ref[:, :C] = q_ref[...]
    out_ref[:, C:] = _build_r(rnum_ref[...], d_ref[...], bmd_ref[...],
                              bacc_ref[...])


def _final(q, rnum, dpart, bmd, bacc):
    return pl.pallas_call(
        _final_body,
        out_shape=jax.ShapeDtypeStruct((B, 2 * C), jnp.float32),
    )(q, rnum, dpart, bmd, bacc)


_MESH = plsc.VectorSubcoreMesh(
    core_axis_name="c", subcore_axis_name="s", num_cores=NC, num_subcores=NS)


def _att_body(x_hbm, q_hbm, seg_hbm, dpart_hbm, rnum_hbm, bmd_hbm, bacc_hbm,
              seg2, qg2, xc2, d_v, accbuf, b0md, b0acc, md_st, idx1,
              shared, sem_x, sem_q):
    cid = lax.axis_index("c")
    sid = lax.axis_index("s")
    wid = sid * NC + cid
    base = wid * RPT
    zero16 = jnp.zeros((L,), jnp.float32)

    def zrow(i, _):
        def zcol(j, _):
            xc2[0, i, pl.ds(j * L, L)] = zero16
            return 0
        lax.fori_loop(0, CL, zcol, 0)
        return 0

    lax.fori_loop(0, 64, zrow, 0)

    @pl.when(sid == 0)
    def _():
        def zshared(j, _):
            pltpu.sync_copy(xc2.at[0, pl.ds(0, 64)],
                            shared.at[pl.ds(j * 64, 64)])
            return 0
        lax.fori_loop(0, B // 64, zshared, 0)

    plsc.subcore_barrier()

    def dinit(i, _):
        d_v[pl.ds(i * L, L)] = zero16
        return 0

    lax.fori_loop(0, (B + L) // L, dinit, 0)

    def zsmall(i, _):
        b0acc[0, pl.ds(i * L, L)] = zero16
        accbuf[0, pl.ds(i * L, L)] = zero16
        return 0

    lax.fori_loop(0, CL, zsmall, 0)
    b0md[pl.ds(0, L)] = zero16
    _store1(b0md, 0, jnp.float32(NEG))
    _store1(b0md, 2, jnp.float32(-1.0))

    pltpu.sync_copy(seg_hbm.at[pl.ds(base, CH)], seg2.at[0])
    pltpu.async_copy(x_hbm.at[pl.ds(base, CH)], xc2.at[0], sem_x)
    pltpu.async_copy(q_hbm.at[seg2.at[0]], qg2.at[0], sem_q)

    def chunk_body(k, carry):
        row0 = base + k * CH
        buf = lax.rem(k, 2)
        nxt = lax.rem(k + 1, 2)
        pltpu.make_async_copy(
            x_hbm.at[pl.ds(row0, CH)], xc2.at[buf], sem_x).wait()
        pltpu.make_async_copy(
            q_hbm.at[seg2.at[buf]], qg2.at[buf], sem_q).wait()

        @pl.when(k + 1 < NCHUNK)
        def _():
            row1 = base + (k + 1) * CH
            pltpu.sync_copy(seg_hbm.at[pl.ds(row1, CH)], seg2.at[nxt])
            pltpu.async_copy(x_hbm.at[pl.ds(row1, CH)], xc2.at[nxt], sem_x)
            pltpu.async_copy(q_hbm.at[seg2.at[nxt]], qg2.at[nxt], sem_q)

        ngrp = jnp.clip(N - row0, 0, CH) // L

        def grp_body(gi, carry):
            cur_seg, m, dd, first_done, *acc = carry
            off = gi * L
            sv = seg2[buf, pl.ds(off, L)]
            for j in range(L):
                s = sv[j]
                rr = off + j

                xr = [xc2[buf, rr, pl.ds(t * L, L)] for t in range(CL)]
                p = [zero16, zero16, zero16, zero16]
                for t in range(CL):
                    p[t % 4] = p[t % 4] + xr[t] * qg2[buf, rr, pl.ds(t * L, L)]
                e = jnp.sum((p[0] + p[1]) + (p[2] + p[3]))

                new = s != cur_seg
                flush = jnp.logical_and(new, cur_seg >= 0)

                @pl.when(flush)
                def _():
                    @pl.when(first_done == 0)
                    def _():
                        for t in range(CL):
                            b0acc[0, pl.ds(t * L, L)] = acc[t]
                        _store1(b0md, 0, m)
                        _store1(b0md, 1, dd)
                        _store1(b0md, 2, cur_seg.astype(jnp.float32))

                    @pl.when(first_done != 0)
                    def _():
                        for t in range(CL):
                            accbuf[0, pl.ds(t * L, L)] = acc[t]
                        _store1(d_v, cur_seg, dd)
                        _store1(idx1, 0, cur_seg)
                        pltpu.sync_copy(accbuf, shared.at[idx1], add=True)

                first_done = jnp.where(flush, 1, first_done)

                m_new = jnp.where(new, e, jnp.maximum(m, e))
                scale = jnp.exp(jnp.broadcast_to(
                    jnp.where(new, NEG, m - m_new), (L,)))
                ee = jnp.exp(jnp.broadcast_to(e - m_new, (L,)))
                dd = jnp.where(new, 1.0, dd * scale[0] + ee[0])
                acc = [acc[t] * scale + ee * xr[t] for t in range(CL)]
                m = m_new
                cur_seg = s
            return (cur_seg, m, dd, first_done, *acc)

        return lax.fori_loop(0, ngrp, grp_body, carry)

    init = (jnp.int32(-1), jnp.float32(NEG), jnp.float32(0.0), jnp.int32(0),
            *([jnp.zeros((L,), jnp.float32)] * CL))
    cur_seg, m, dd, first_done, *acc = lax.fori_loop(
        0, NCHUNK, chunk_body, init)

    pltpu.sync_copy(b0md, bmd_hbm.at[2 * wid])
    pltpu.sync_copy(b0acc, bacc_hbm.at[pl.ds(2 * wid, 1)])

    for t in range(CL):
        accbuf[0, pl.ds(t * L, L)] = acc[t]
    md_st[pl.ds(0, L)] = zero16
    _store1(md_st, 0, m)
    _store1(md_st, 1, dd)
    _store1(md_st, 2, cur_seg.astype(jnp.float32))
    pltpu.sync_copy(md_st, bmd_hbm.at[2 * wid + 1])
    pltpu.sync_copy(accbuf, bacc_hbm.at[pl.ds(2 * wid + 1, 1)])

    pltpu.sync_copy(d_v.at[pl.ds(0, B)], dpart_hbm.at[wid])
    plsc.subcore_barrier()

    @pl.when(sid == 0)
    def _():
        pltpu.sync_copy(shared, rnum_hbm.at[cid])


@functools.partial(
    pl.kernel,
    out_type=[
        jax.ShapeDtypeStruct((NW, B), jnp.float32),
        jax.ShapeDtypeStruct((NC, B, C), jnp.float32),
        jax.ShapeDtypeStruct((NB, L), jnp.float32),
        jax.ShapeDtypeStruct((NB, C), jnp.float32),
    ],
    mesh=_MESH,
    compiler_params=pltpu.CompilerParams(
        use_tc_tiling_on_sc=False, needs_layout_passes=False),
    scratch_types=[
        pltpu.VMEM((2, CH), jnp.int32),
        pltpu.VMEM((2, CH, C), jnp.float32),
        pltpu.VMEM((2, CH, C), jnp.float32),
        pltpu.VMEM((B + L,), jnp.float32),
        pltpu.VMEM((1, C), jnp.float32),
        pltpu.VMEM((L,), jnp.float32),
        pltpu.VMEM((1, C), jnp.float32),
        pltpu.VMEM((L,), jnp.float32),
        pltpu.VMEM((1,), jnp.int32),
        pltpu.VMEM_SHARED((B, C), jnp.float32),
        pltpu.SemaphoreType.DMA,
        pltpu.SemaphoreType.DMA,
    ],
)
def _att_kernel(x_hbm, q_hbm, seg_hbm, dpart_hbm, rnum_hbm, bmd_hbm,
                bacc_hbm, *scratch):
    _att_body(x_hbm, q_hbm, seg_hbm, dpart_hbm, rnum_hbm, bmd_hbm, bacc_hbm,
              *scratch)


def kernel(x, batch, W_ih, W_hh, b_ih, b_hh):
    seg = batch.astype(jnp.int32)
    segp = jnp.pad(seg, (0, NP - N))
    xp = jnp.pad(x, ((0, NP - N), (0, 0)))
    wih_t = W_ih.T
    whh_t = W_hh.T
    bias = (b_ih + b_hh)[None, :]

    q = jnp.zeros((B, C), jnp.float32)
    c = jnp.zeros((B, C), jnp.float32)
    rnum = jnp.zeros((NC, B, C), jnp.float32)
    dpart = jnp.zeros((NW, B), jnp.float32)
    bmd = jnp.zeros((NB, L), jnp.float32).at[:, 2].set(-1.0)
    bacc = jnp.zeros((NB, C), jnp.float32)

    for _ in range(STEPS):
        q, c = _lstm_step(q, rnum, dpart, bmd, bacc, c, wih_t, whh_t, bias)
        dpart, rnum, bmd, bacc = _att_kernel(xp, q, segp)

    return _final(q, rnum, dpart, bmd, bacc)
